# Initial kernel scaffold; baseline (speedup 1.0000x reference)
import jax
import jax.numpy as jnp
from jax.experimental import pallas as pl


def kernel(batch, A_indices, A_values, user_emb, item_emb):
    raise NotImplementedError("write your pallas kernel here")


# speedup vs baseline: 2.1525x; 2.1525x over previous
import functools
import jax
import jax.numpy as jnp
from jax import lax
from jax.experimental import pallas as pl
from jax.experimental.pallas import tpu as pltpu
from jax.experimental.pallas import tpu_sc as plsc

NUSERS = 30000
NNODES = 50000
D = 64
NEDGES = 800000
B = 4096

NC = 2
NS = 16
NW = NC * NS

HALF = NNODES // NC
ACC_ROWS = 25600
ZROWS = ACC_ROWS // NS

EPAD = 819200
CHUNK = 2048
SUB = 128
NSUB = CHUNK // SUB
PER_TILE = EPAD // NS
NBIG = PER_TILE // CHUNK

BPT = B // NW

_mesh = plsc.VectorSubcoreMesh(core_axis_name="c", subcore_axis_name="s")

_GATHER_DN = lax.GatherDimensionNumbers(
    offset_dims=(), collapsed_slice_dims=(0,), start_index_map=(0,))


def _bcast_lane(v16, lane):
    return lax.gather(v16, jnp.full((16, 1), lane, jnp.int32), _GATHER_DN,
                      slice_sizes=(1,),
                      mode=lax.GatherScatterMode.PROMISE_IN_BOUNDS)


def _shuffle(v16, idx16):
    return lax.gather(v16, idx16[:, None], _GATHER_DN, slice_sizes=(1,),
                      mode=lax.GatherScatterMode.PROMISE_IN_BOUNDS)


def _lane_reduce_sum(v16, lanes):
    for sh in (8, 4, 2, 1):
        v16 = v16 + _shuffle(v16, lanes ^ sh)
    return v16


@functools.partial(
    pl.kernel,
    out_type=jax.ShapeDtypeStruct((NNODES, D), jnp.float32),
    mesh=_mesh,
    compiler_params=pltpu.CompilerParams(use_tc_tiling_on_sc=False),
    scratch_types=[
        pltpu.VMEM((CHUNK,), jnp.int32),
        pltpu.VMEM((CHUNK,), jnp.float32),
        pltpu.VMEM((CHUNK,), jnp.int32),
        pltpu.VMEM((NSUB, SUB), jnp.int32),
        pltpu.VMEM((SUB, D), jnp.float32),
        pltpu.VMEM_SHARED((ACC_ROWS, D), jnp.float32),
        pltpu.SemaphoreType.DMA,
    ],
)
def _spmm(x_hbm, row_hbm, col_hbm, val_hbm, zeros_hbm, y_hbm,
          col_v, val_v, row_v, lrow_v, g_v, acc, sem):
    c = lax.axis_index("c")
    s = lax.axis_index("s")
    row_off = c * HALF

    pltpu.sync_copy(zeros_hbm, acc.at[pl.ds(s * ZROWS, ZROWS)])
    plsc.subcore_barrier()

    ebase = s * PER_TILE

    def big_iter(b, carry):
        off = ebase + b * CHUNK
        pltpu.sync_copy(col_hbm.at[pl.ds(off, CHUNK)], col_v)
        pltpu.sync_copy(val_hbm.at[pl.ds(off, CHUNK)], val_v)
        pltpu.sync_copy(row_hbm.at[pl.ds(off, CHUNK)], row_v)

        def lrow_iter(j, carry2):
            for r in range(SUB // 16):
                rv = row_v[pl.ds(j * SUB + r * 16, 16)]
                lv = rv - row_off
                ok = (lv >= 0) & (lv < HALF)
                lrow_v[j, pl.ds(r * 16, 16)] = jnp.where(ok, lv, HALF)
            return carry2
        lax.fori_loop(0, NSUB, lrow_iter, 0)

        def sub_iter(j, carry2):
            pltpu.async_copy(
                x_hbm.at[col_v.at[pl.ds(j * SUB, SUB)]], g_v, sem).wait()

            def grp_iter(gi, carry3):
                vv16 = val_v[pl.ds(j * SUB + gi * 16, 16)]
                for l in range(16):
                    vv = _bcast_lane(vv16, l)
                    e = gi * 16 + l
                    for q in range(D // 16):
                        g_v[e, pl.ds(q * 16, 16)] = (
                            g_v[e, pl.ds(q * 16, 16)] * vv)
                return carry3
            lax.fori_loop(0, SUB // 16, grp_iter, 0)

            pltpu.sync_copy(g_v, acc.at[lrow_v.at[j]], add=True)
            return carry2
        lax.fori_loop(0, NSUB, sub_iter, 0)
        return carry
    lax.fori_loop(0, NBIG, big_iter, 0)

    plsc.subcore_barrier()

    @pl.when(s < NS - 1)
    def _wb():
        pltpu.sync_copy(acc.at[pl.ds(s * ZROWS, ZROWS)],
                        y_hbm.at[pl.ds(row_off + s * ZROWS, ZROWS)])

    @pl.when(s == NS - 1)
    def _wb_last():
        tail = HALF - (NS - 1) * ZROWS
        pltpu.sync_copy(acc.at[pl.ds((NS - 1) * ZROWS, tail)],
                        y_hbm.at[pl.ds(row_off + (NS - 1) * ZROWS, tail)])


@functools.partial(
    pl.kernel,
    out_type=jax.ShapeDtypeStruct((B,), jnp.float32),
    mesh=_mesh,
    compiler_params=pltpu.CompilerParams(use_tc_tiling_on_sc=False),
    scratch_types=[
        pltpu.VMEM((BPT,), jnp.int32),
        pltpu.VMEM((BPT,), jnp.int32),
        pltpu.VMEM((4, BPT, D), jnp.float32),
        pltpu.VMEM((4, BPT, D), jnp.float32),
        pltpu.VMEM((BPT,), jnp.float32),
        pltpu.SemaphoreType.DMA,
    ],
)
def _score(x0, x1, x2, x3, ui_hbm, ii_hbm, out_hbm,
           ub, ib, gu, gi, ob, sem):
    c = lax.axis_index("c")
    s = lax.axis_index("s")
    w = s * NC + c
    base = w * BPT

    pltpu.sync_copy(ui_hbm.at[pl.ds(base, BPT)], ub)
    pltpu.sync_copy(ii_hbm.at[pl.ds(base, BPT)], ib)
    for t, x in enumerate((x0, x1, x2, x3)):
        pltpu.async_copy(x.at[ub], gu.at[t], sem).wait()
        pltpu.async_copy(x.at[ib], gi.at[t], sem).wait()

    lanes = lax.iota(jnp.int32, 16)

    def grp(g_idx, carry):
        pack = jnp.zeros((16,), jnp.float32)
        for l in range(16):
            e = g_idx * 16 + l
            acc = jnp.zeros((16,), jnp.float32)
            for q in range(D // 16):
                uq = (gu[0, e, pl.ds(q * 16, 16)] + gu[1, e, pl.ds(q * 16, 16)]
                      + gu[2, e, pl.ds(q * 16, 16)]
                      + gu[3, e, pl.ds(q * 16, 16)])
                iq = (gi[0, e, pl.ds(q * 16, 16)] + gi[1, e, pl.ds(q * 16, 16)]
                      + gi[2, e, pl.ds(q * 16, 16)]
                      + gi[3, e, pl.ds(q * 16, 16)])
                acc = acc + uq * iq
            red = _lane_reduce_sum(acc, lanes) * jnp.float32(1.0 / 16.0)
            pack = jnp.where(lanes == l, red, pack)
        ob[pl.ds(g_idx * 16, 16)] = pack
        return carry
    lax.fori_loop(0, BPT // 16, grp, 0)

    pltpu.sync_copy(ob, out_hbm.at[pl.ds(base, BPT)])


def kernel(batch, A_indices, A_values, user_emb, item_emb):
    x0 = jnp.concatenate([user_emb, item_emb], axis=0)
    pad = EPAD - NEDGES
    row = jnp.concatenate([A_indices[0], jnp.zeros((pad,), jnp.int32)])
    col = jnp.concatenate([A_indices[1], jnp.zeros((pad,), jnp.int32)])
    val = jnp.concatenate([A_values, jnp.zeros((pad,), jnp.float32)])
    zeros = jnp.zeros((ZROWS, D), jnp.float32)

    x1 = _spmm(x0, row, col, val, zeros)
    x2 = _spmm(x1, row, col, val, zeros)
    x3 = _spmm(x2, row, col, val, zeros)

    ui = batch[:, 0]
    ii = batch[:, 1] + NUSERS
    return _score(x0, x1, x2, x3, ui, ii)
